# baseline (speedup 1.0000x reference)
import functools

import jax
import jax.numpy as jnp
from jax import lax
from jax.experimental import pallas as pl
from jax.experimental.pallas import tpu as pltpu
from jax.experimental.pallas import tpu_sc as plsc

N = 10000
E = 320000
IN_DIM = 128
HID = 256
OUT = 256
HALF = OUT // 2

NC = 2
NS = 16
NW = NC * NS
L = 16

ROW_BLK = 2000
CH = 80

_f32 = jnp.float32


def _tables_body(ent_ref, rel_ref, W_ent_ref, b_ent_ref, W_rel_ref, b_rel_ref,
                 W_rel2_ref, b_rel2_ref, W_a_ref, b_a_ref, W_fc_ref, b_fc_ref,
                 h0_ref, bh_ref, rh_ref, t2_ref, a3_ref):
    ent_proj = jnp.dot(ent_ref[...], W_ent_ref[...],
                       preferred_element_type=_f32) + b_ent_ref[...]
    rel_proj = jnp.dot(rel_ref[...], W_rel_ref[...],
                       preferred_element_type=_f32) + b_rel_ref[...]

    Wfs = W_fc_ref[0:HID, :]
    Wfd = W_fc_ref[HID:2 * HID, :]
    Wfr = W_fc_ref[2 * HID:3 * HID, :]

    A = jnp.dot(ent_proj, Wfs, preferred_element_type=_f32)
    B = jnp.dot(ent_proj, Wfd, preferred_element_type=_f32)
    R = jnp.dot(rel_proj, Wfr, preferred_element_type=_f32)
    T2 = jnp.dot(rel_proj, W_rel2_ref[...],
                 preferred_element_type=_f32) + b_rel2_ref[...]

    h0_ref[...] = A + b_fc_ref[...]
    bh_ref[0] = B[:, :HALF]
    bh_ref[1] = B[:, HALF:]
    rh_ref[0] = R[:, :HALF]
    rh_ref[1] = R[:, HALF:]
    t2_ref[...] = T2

    wa = W_a_ref[...]
    c0 = jnp.sum(b_fc_ref[...] * wa) + b_a_ref[0, 0]
    a_s = jnp.sum(A * wa, axis=1) + c0
    a_d = jnp.sum(B * wa, axis=1)
    a_r = jnp.sum(R * wa, axis=1)
    z = jnp.zeros_like(a_s)
    a3_ref[...] = jnp.stack([a_s, a_d, a_r, z, z, z, z, z], axis=1)


def _tables(ent_embed, rel_embed, W_ent, b_ent, W_rel, b_rel, W_rel2, b_rel2,
            W_a, b_a, W_fc, b_fc):
    grid = (N // ROW_BLK,)
    full = lambda shape: pl.BlockSpec(shape, lambda i: tuple(0 for _ in shape))
    out = pl.pallas_call(
        _tables_body,
        grid=grid,
        in_specs=[
            pl.BlockSpec((ROW_BLK, IN_DIM), lambda i: (i, 0)),
            pl.BlockSpec((ROW_BLK, IN_DIM), lambda i: (i, 0)),
            full((IN_DIM, HID)), full((1, HID)),
            full((IN_DIM, HID)), full((1, HID)),
            full((HID, OUT)), full((1, OUT)),
            full((1, OUT)), full((1, 1)),
            full((3 * HID, OUT)), full((1, OUT)),
        ],
        out_specs=[
            pl.BlockSpec((ROW_BLK, OUT), lambda i: (i, 0)),
            pl.BlockSpec((NC, ROW_BLK, HALF), lambda i: (0, i, 0)),
            pl.BlockSpec((NC, ROW_BLK, HALF), lambda i: (0, i, 0)),
            pl.BlockSpec((ROW_BLK, OUT), lambda i: (i, 0)),
            pl.BlockSpec((ROW_BLK, 8), lambda i: (i, 0)),
        ],
        out_shape=[
            jax.ShapeDtypeStruct((N, OUT), _f32),
            jax.ShapeDtypeStruct((NC, N, HALF), _f32),
            jax.ShapeDtypeStruct((NC, N, HALF), _f32),
            jax.ShapeDtypeStruct((N, OUT), _f32),
            jax.ShapeDtypeStruct((N, 8), _f32),
        ],
    )(ent_embed, rel_embed, W_ent, b_ent.reshape(1, HID), W_rel,
      b_rel.reshape(1, HID), W_rel2, b_rel2.reshape(1, OUT),
      W_a.reshape(1, OUT), b_a.reshape(1, 1), W_fc, b_fc.reshape(1, OUT))
    return out


_EPT1 = E // NW
_NCH1 = _EPT1 // CH


def _sc1_body(src_hbm, dst_hbm, rel_hbm, as_hbm, ad_hbm, ar_hbm, t2_hbm,
              b_hbm, bnode2_hbm, relout_hbm,
              as_v, ad_v, ar_v,
              srcbuf0, srcbuf1, dstbuf0, dstbuf1, relbuf0, relbuf1,
              bbuf0, bbuf1, zbuf, ibuf0, ibuf1, Tbuf0, Tbuf1,
              isem0, isem1, bsem0, bsem1, risem0, risem1,
              gsem0, gsem1, wsem0, wsem1,
              acc_sh):
    cid = lax.axis_index("c")
    sid = lax.axis_index("s")
    wid = cid * NS + sid
    base = wid * _EPT1
    srcbuf = (srcbuf0, srcbuf1)
    dstbuf = (dstbuf0, dstbuf1)
    relbuf = (relbuf0, relbuf1)
    bbuf = (bbuf0, bbuf1)
    ibuf = (ibuf0, ibuf1)
    Tbuf = (Tbuf0, Tbuf1)
    isem = (isem0, isem1)
    bsem = (bsem0, bsem1)
    risem = (risem0, risem1)
    gsem = (gsem0, gsem1)
    wsem = (wsem0, wsem1)

    pltpu.sync_copy(as_hbm, as_v)
    pltpu.sync_copy(ad_hbm, ad_v)
    pltpu.sync_copy(ar_hbm, ar_v)

    @pl.when(sid == 0)
    def _():
        @pl.loop(0, 2000 // L)
        def _z(i):
            zbuf[pl.ds(i * L, L)] = jnp.zeros((L,), _f32)

        for j in range(N // 2000):
            pltpu.sync_copy(zbuf, acc_sh.at[pl.ds(j * 2000, 2000)])

    plsc.subcore_barrier()

    def fire_sidx(p, c):
        off = base + c * CH
        pltpu.async_copy(src_hbm.at[pl.ds(off, CH)], srcbuf[p], isem[p])
        pltpu.async_copy(dst_hbm.at[pl.ds(off, CH)], dstbuf[p], isem[p])
        pltpu.async_copy(rel_hbm.at[pl.ds(off, CH)], relbuf[p], isem[p])

    def wait_sidx(p, c):
        off = base + c * CH
        pltpu.make_async_copy(src_hbm.at[pl.ds(off, CH)], srcbuf[p],
                              isem[p]).wait()
        pltpu.make_async_copy(dst_hbm.at[pl.ds(off, CH)], dstbuf[p],
                              isem[p]).wait()
        pltpu.make_async_copy(rel_hbm.at[pl.ds(off, CH)], relbuf[p],
                              isem[p]).wait()

    def wait_bwrite(p, c):
        off = base + c * CH
        pltpu.make_async_copy(bbuf[p], b_hbm.at[pl.ds(off, CH)],
                              bsem[p]).wait()

    def fire_ridx(p, c):
        pltpu.async_copy(rel_hbm.at[pl.ds(base + c * CH, CH)],
                         ibuf[p], risem[p])

    def wait_ridx(p, c):
        pltpu.make_async_copy(rel_hbm.at[pl.ds(base + c * CH, CH)],
                              ibuf[p], risem[p]).wait()

    def wait_rwrite(p, c):
        pltpu.make_async_copy(Tbuf[p],
                              relout_hbm.at[pl.ds(base + c * CH, CH)],
                              wsem[p]).wait()

    def handle(p, c, last):
        np_ = 1 - p
        if not last:
            wait_ridx(np_, c + 1)

            @pl.when(c >= 1)
            def _():
                wait_rwrite(np_, c - 1)

            pltpu.async_copy(t2_hbm.at[ibuf[np_]], Tbuf[np_], gsem[np_])
            wait_sidx(np_, c + 1)

        @pl.when(c >= 2)
        def _():
            wait_bwrite(p, c - 2)

        @pl.loop(0, CH // L)
        def _vec(i):
            sl = pl.ds(i * L, L)
            sc = (plsc.load_gather(as_v, [srcbuf[p][sl]])
                  + plsc.load_gather(ad_v, [dstbuf[p][sl]])
                  + plsc.load_gather(ar_v, [relbuf[p][sl]]))
            bbuf[p][sl] = jnp.exp(jnp.where(sc >= 0.0, sc, sc * 0.01))

        pltpu.async_copy(bbuf[p], b_hbm.at[pl.ds(base + c * CH, CH)], bsem[p])
        pltpu.sync_copy(bbuf[p], acc_sh.at[srcbuf[p]], add=True)

        @pl.when(c + 2 < _NCH1)
        def _():
            fire_sidx(p, c + 2)

        pltpu.make_async_copy(t2_hbm.at[ibuf[p]], Tbuf[p], gsem[p]).wait()
        if last:
            pltpu.sync_copy(Tbuf[p], relout_hbm.at[pl.ds(base + c * CH, CH)])
        else:
            pltpu.async_copy(Tbuf[p],
                             relout_hbm.at[pl.ds(base + c * CH, CH)], wsem[p])

            @pl.when(c + 2 < _NCH1)
            def _():
                fire_ridx(p, c + 2)

    fire_sidx(0, 0)
    fire_ridx(0, 0)
    wait_sidx(0, 0)
    wait_ridx(0, 0)
    pltpu.async_copy(t2_hbm.at[ibuf0], Tbuf0, gsem0)
    fire_sidx(1, 1)
    fire_ridx(1, 1)

    @pl.loop(0, _NCH1 - 1, step=2)
    def _pair(ci):
        handle(0, ci, False)
        handle(1, ci + 1, False)

    c_last = _NCH1 - 1
    wait_rwrite(1, c_last - 1)
    handle(0, c_last, True)
    wait_bwrite(1, c_last - 1)
    wait_bwrite(0, c_last)

    plsc.subcore_barrier()

    @pl.when(sid == 0)
    def _():
        for j in range(N // 2000):
            pltpu.sync_copy(acc_sh.at[pl.ds(j * 2000, 2000)], zbuf)
            pltpu.sync_copy(zbuf, bnode2_hbm.at[pl.ds(cid * N + j * 2000, 2000)])


def _sc1(src, dst, rel, a_s, a_d, a_r, t2):
    mesh = plsc.VectorSubcoreMesh(core_axis_name="c", subcore_axis_name="s")
    f = pl.kernel(
        _sc1_body,
        out_type=[
            jax.ShapeDtypeStruct((E,), _f32),
            jax.ShapeDtypeStruct((NC * N,), _f32),
            jax.ShapeDtypeStruct((E, OUT), _f32),
        ],
        mesh=mesh,
        scratch_types=[
            pltpu.VMEM((N,), _f32),
            pltpu.VMEM((N,), _f32),
            pltpu.VMEM((N,), _f32),
            pltpu.VMEM((CH,), jnp.int32),
            pltpu.VMEM((CH,), jnp.int32),
            pltpu.VMEM((CH,), jnp.int32),
            pltpu.VMEM((CH,), jnp.int32),
            pltpu.VMEM((CH,), jnp.int32),
            pltpu.VMEM((CH,), jnp.int32),
            pltpu.VMEM((CH,), _f32),
            pltpu.VMEM((CH,), _f32),
            pltpu.VMEM((2000,), _f32),
            pltpu.VMEM((CH,), jnp.int32),
            pltpu.VMEM((CH,), jnp.int32),
            pltpu.VMEM((CH, OUT), _f32),
            pltpu.VMEM((CH, OUT), _f32),
            pltpu.SemaphoreType.DMA,
            pltpu.SemaphoreType.DMA,
            pltpu.SemaphoreType.DMA,
            pltpu.SemaphoreType.DMA,
            pltpu.SemaphoreType.DMA,
            pltpu.SemaphoreType.DMA,
            pltpu.SemaphoreType.DMA,
            pltpu.SemaphoreType.DMA,
            pltpu.SemaphoreType.DMA,
            pltpu.SemaphoreType.DMA,
            pltpu.VMEM_SHARED((N,), _f32),
        ],
        compiler_params=pltpu.CompilerParams(needs_layout_passes=False),
    )
    return f(src, dst, rel, a_s, a_d, a_r, t2)


_EPT2 = E // NS
_NCH2 = _EPT2 // CH
_WBT = 10
_RPT = N // _WBT
_WB = 40


def _sc2_body(src_hbm, dst_hbm, rel_hbm, b_hbm, bflat_hbm, rflat_hbm,
              s_hbm,
              srcbuf0, srcbuf1, dstbuf0, dstbuf1, relbuf0, relbuf1,
              bbuf0, bbuf1, sbuf0, sbuf1,
              Bbuf0, Bbuf1, Rbuf0, Rbuf1,
              isem0, isem1, Bsem0, Bsem1, Rsem0, Rsem1, ssem0, ssem1,
              acc_sh):
    cid = lax.axis_index("c")
    sid = lax.axis_index("s")
    srcbuf = (srcbuf0, srcbuf1)
    dstbuf = (dstbuf0, dstbuf1)
    relbuf = (relbuf0, relbuf1)
    bbuf = (bbuf0, bbuf1)
    sbuf = (sbuf0, sbuf1)
    Bbuf = (Bbuf0, Bbuf1)
    Rbuf = (Rbuf0, Rbuf1)
    isem = (isem0, isem1)
    Bsem = (Bsem0, Bsem1)
    Rsem = (Rsem0, Rsem1)
    ssem = (ssem0, ssem1)
    tab_off = cid * N
    ebase = sid * _EPT2

    @pl.when(sid < _WBT)
    def _():
        @pl.loop(0, _WB)
        def _z(r):
            for j in range(HALF // L):
                Bbuf0[r, pl.ds(j * L, L)] = jnp.zeros((L,), _f32)

        for j in range(_RPT // _WB):
            pltpu.sync_copy(Bbuf0.at[pl.ds(0, _WB)],
                            acc_sh.at[pl.ds(sid * _RPT + j * _WB, _WB)])

    plsc.subcore_barrier()

    def fire_idx(p, c):
        off = ebase + c * CH
        pltpu.async_copy(src_hbm.at[pl.ds(off, CH)], srcbuf[p], isem[p])
        pltpu.async_copy(dst_hbm.at[pl.ds(off, CH)], dstbuf[p], isem[p])
        pltpu.async_copy(rel_hbm.at[pl.ds(off, CH)], relbuf[p], isem[p])
        pltpu.async_copy(b_hbm.at[pl.ds(off, CH)], bbuf[p], isem[p])

    def wait_idx(p, c):
        off = ebase + c * CH
        pltpu.make_async_copy(src_hbm.at[pl.ds(off, CH)], srcbuf[p],
                              isem[p]).wait()
        pltpu.make_async_copy(dst_hbm.at[pl.ds(off, CH)], dstbuf[p],
                              isem[p]).wait()
        pltpu.make_async_copy(rel_hbm.at[pl.ds(off, CH)], relbuf[p],
                              isem[p]).wait()
        pltpu.make_async_copy(b_hbm.at[pl.ds(off, CH)], bbuf[p],
                              isem[p]).wait()

    def fire_gather(p):
        @pl.loop(0, CH // L)
        def _mkidx(i):
            sl = pl.ds(i * L, L)
            dstbuf[p][sl] = dstbuf[p][sl] + tab_off
            relbuf[p][sl] = relbuf[p][sl] + tab_off

        pltpu.async_copy(bflat_hbm.at[dstbuf[p]], Bbuf[p], Bsem[p])
        pltpu.async_copy(rflat_hbm.at[relbuf[p]], Rbuf[p], Rsem[p])

    def wait_gather(p):
        pltpu.make_async_copy(bflat_hbm.at[dstbuf[p]], Bbuf[p],
                              Bsem[p]).wait()
        pltpu.make_async_copy(rflat_hbm.at[relbuf[p]], Rbuf[p],
                              Rsem[p]).wait()

    def wait_scatter(p):
        pltpu.make_async_copy(Bbuf[p], acc_sh.at[sbuf[p]], ssem[p]).wait()

    def consume(p):
        @pl.loop(0, CH)
        def _scale(r):
            wb = plsc.load_gather(bbuf[p], [jnp.full((L,), r, jnp.int32)])
            for j in range(HALF // L):
                sl = pl.ds(j * L, L)
                Bbuf[p][r, sl] = wb * (Bbuf[p][r, sl] + Rbuf[p][r, sl])

        @pl.loop(0, CH // L)
        def _cpy(i):
            sl = pl.ds(i * L, L)
            sbuf[p][sl] = srcbuf[p][sl]

        pltpu.async_copy(Bbuf[p], acc_sh.at[sbuf[p]], ssem[p], add=True)

    def handle(p, c, has_next=True, has_next2=True):
        np_ = 1 - p
        if has_next:
            wait_idx(np_, c + 1)

            @pl.when(c >= 1)
            def _():
                wait_scatter(np_)

            fire_gather(np_)
        else:
            wait_scatter(np_)
        wait_gather(p)
        consume(p)
        if has_next2:
            fire_idx(p, c + 2)

    fire_idx(0, 0)
    wait_idx(0, 0)
    fire_gather(0)
    fire_idx(1, 1)

    @pl.loop(0, _NCH2 - 2, step=2)
    def _pair(ci):
        handle(0, ci)
        handle(1, ci + 1)

    handle(0, _NCH2 - 2, has_next2=False)
    handle(1, _NCH2 - 1, has_next=False, has_next2=False)
    wait_scatter(1)

    plsc.subcore_barrier()

    @pl.when(sid < _WBT)
    def _():
        for j in range(_RPT // _WB):
            row = sid * _RPT + j * _WB
            pltpu.sync_copy(acc_sh.at[pl.ds(row, _WB)], Bbuf0.at[pl.ds(0, _WB)])
            pltpu.sync_copy(Bbuf0.at[pl.ds(0, _WB)],
                            s_hbm.at[cid, pl.ds(row, _WB)])


def _sc2(src, dst, rel, b, bflat, rflat):
    mesh = plsc.VectorSubcoreMesh(core_axis_name="c", subcore_axis_name="s")
    f = pl.kernel(
        _sc2_body,
        out_type=[
            jax.ShapeDtypeStruct((NC, N, HALF), _f32),
        ],
        mesh=mesh,
        scratch_types=(
            [pltpu.VMEM((CH,), jnp.int32)] * 6
            + [pltpu.VMEM((CH,), _f32)] * 2
            + [pltpu.VMEM((CH,), jnp.int32)] * 2
            + [pltpu.VMEM((CH, HALF), _f32)] * 4
            + [pltpu.SemaphoreType.DMA] * 8
            + [pltpu.VMEM_SHARED((N, HALF), _f32)]
        ),
        compiler_params=pltpu.CompilerParams(needs_layout_passes=False),
    )
    return f(src, dst, rel, b, bflat, rflat)[0]


def _fin_body(h0_ref, s0_ref, s1_ref, bn0_ref, bn1_ref, h_ref):
    bn = bn0_ref[...] + bn1_ref[...]
    s = jnp.concatenate([s0_ref[0], s1_ref[0]], axis=1)
    h_ref[...] = jnp.where(bn > 0.0,
                           h0_ref[...] + s / jnp.maximum(bn, 1e-30), 0.0)


def _finalize(h0, s, bnode2):
    nb = N // ROW_BLK
    bn_r = bnode2.reshape(NC * N, 1)
    return pl.pallas_call(
        _fin_body,
        grid=(nb,),
        in_specs=[
            pl.BlockSpec((ROW_BLK, OUT), lambda i: (i, 0)),
            pl.BlockSpec((1, ROW_BLK, HALF), lambda i: (0, i, 0)),
            pl.BlockSpec((1, ROW_BLK, HALF), lambda i: (1, i, 0)),
            pl.BlockSpec((ROW_BLK, 1), lambda i: (i, 0)),
            pl.BlockSpec((ROW_BLK, 1), lambda i: (nb + i, 0)),
        ],
        out_specs=pl.BlockSpec((ROW_BLK, OUT), lambda i: (i, 0)),
        out_shape=jax.ShapeDtypeStruct((N, OUT), _f32),
    )(h0, s, s, bn_r, bn_r)


def kernel(triplets, ent_embed, rel_embed, W_ent, b_ent, W_rel, b_rel,
           W_rel2, b_rel2, W_a, b_a, W_fc, b_fc):
    src = triplets[:, 0]
    dst = triplets[:, 1]
    rel = triplets[:, 2]

    h0, bh, rh, t2, a3 = _tables(ent_embed, rel_embed, W_ent, b_ent, W_rel,
                                 b_rel, W_rel2, b_rel2, W_a, b_a, W_fc, b_fc)

    b, bnode2, rel_out = _sc1(src, dst, rel, a3[:, 0], a3[:, 1], a3[:, 2], t2)

    s = _sc2(src, dst, rel, b,
             bh.reshape(NC * N, HALF), rh.reshape(NC * N, HALF))

    h = _finalize(h0, s, bnode2)
    return (h, rel_out)


# speedup vs baseline: 1.3252x; 1.3252x over previous
import functools

import jax
import jax.numpy as jnp
from jax import lax
from jax.experimental import pallas as pl
from jax.experimental.pallas import tpu as pltpu
from jax.experimental.pallas import tpu_sc as plsc

N = 10000
E = 320000
IN_DIM = 128
HID = 256
OUT = 256
HALF = OUT // 2

NC = 2
NS = 16
NW = NC * NS
L = 16

ROW_BLK = 2000
CH = 80

_f32 = jnp.float32


def _tables_body(ent_ref, rel_ref, W_ent_ref, b_ent_ref, W_rel_ref, b_rel_ref,
                 W_rel2_ref, b_rel2_ref, W_a_ref, b_a_ref, W_fc_ref, b_fc_ref,
                 h0_ref, bh_ref, rh_ref, t2_ref, a3_ref):
    ent_proj = jnp.dot(ent_ref[...], W_ent_ref[...],
                       preferred_element_type=_f32) + b_ent_ref[...]
    rel_proj = jnp.dot(rel_ref[...], W_rel_ref[...],
                       preferred_element_type=_f32) + b_rel_ref[...]

    Wfs = W_fc_ref[0:HID, :]
    Wfd = W_fc_ref[HID:2 * HID, :]
    Wfr = W_fc_ref[2 * HID:3 * HID, :]

    A = jnp.dot(ent_proj, Wfs, preferred_element_type=_f32)
    B = jnp.dot(ent_proj, Wfd, preferred_element_type=_f32)
    R = jnp.dot(rel_proj, Wfr, preferred_element_type=_f32)
    T2 = jnp.dot(rel_proj, W_rel2_ref[...],
                 preferred_element_type=_f32) + b_rel2_ref[...]

    h0_ref[...] = A + b_fc_ref[...]
    bh_ref[0] = B[:, :HALF]
    bh_ref[1] = B[:, HALF:]
    rh_ref[0] = R[:, :HALF]
    rh_ref[1] = R[:, HALF:]
    t2_ref[...] = T2

    wa = W_a_ref[...]
    c0 = jnp.sum(b_fc_ref[...] * wa) + b_a_ref[0, 0]
    a_s = jnp.sum(A * wa, axis=1) + c0
    a_d = jnp.sum(B * wa, axis=1)
    a_r = jnp.sum(R * wa, axis=1)
    z = jnp.zeros_like(a_s)
    a3_ref[...] = jnp.stack([a_s, a_d, a_r, z, z, z, z, z], axis=1)


def _tables(ent_embed, rel_embed, W_ent, b_ent, W_rel, b_rel, W_rel2, b_rel2,
            W_a, b_a, W_fc, b_fc):
    grid = (N // ROW_BLK,)
    full = lambda shape: pl.BlockSpec(shape, lambda i: tuple(0 for _ in shape))
    out = pl.pallas_call(
        _tables_body,
        grid=grid,
        in_specs=[
            pl.BlockSpec((ROW_BLK, IN_DIM), lambda i: (i, 0)),
            pl.BlockSpec((ROW_BLK, IN_DIM), lambda i: (i, 0)),
            full((IN_DIM, HID)), full((1, HID)),
            full((IN_DIM, HID)), full((1, HID)),
            full((HID, OUT)), full((1, OUT)),
            full((1, OUT)), full((1, 1)),
            full((3 * HID, OUT)), full((1, OUT)),
        ],
        out_specs=[
            pl.BlockSpec((ROW_BLK, OUT), lambda i: (i, 0)),
            pl.BlockSpec((NC, ROW_BLK, HALF), lambda i: (0, i, 0)),
            pl.BlockSpec((NC, ROW_BLK, HALF), lambda i: (0, i, 0)),
            pl.BlockSpec((ROW_BLK, OUT), lambda i: (i, 0)),
            pl.BlockSpec((ROW_BLK, 8), lambda i: (i, 0)),
        ],
        out_shape=[
            jax.ShapeDtypeStruct((N, OUT), _f32),
            jax.ShapeDtypeStruct((NC, N, HALF), _f32),
            jax.ShapeDtypeStruct((NC, N, HALF), _f32),
            jax.ShapeDtypeStruct((N, OUT), _f32),
            jax.ShapeDtypeStruct((N, 8), _f32),
        ],
    )(ent_embed, rel_embed, W_ent, b_ent.reshape(1, HID), W_rel,
      b_rel.reshape(1, HID), W_rel2, b_rel2.reshape(1, OUT),
      W_a.reshape(1, OUT), b_a.reshape(1, 1), W_fc, b_fc.reshape(1, OUT))
    return out


_EPT1 = E // NW
_NCH1 = _EPT1 // CH


def _sc1_body(src_hbm, dst_hbm, rel_hbm, as_hbm, ad_hbm, ar_hbm, t2_hbm,
              b_hbm, bnode2_hbm, relout_hbm,
              as_v, ad_v, ar_v,
              srcbuf0, srcbuf1, dstbuf0, dstbuf1, relbuf0, relbuf1,
              bbuf0, bbuf1, zbuf, ibuf0, ibuf1, Tbuf0, Tbuf1,
              isem0, isem1, bsem0, bsem1, risem0, risem1,
              gsem0, gsem1, wsem0, wsem1,
              acc_sh):
    cid = lax.axis_index("c")
    sid = lax.axis_index("s")
    wid = cid * NS + sid
    base = wid * _EPT1
    srcbuf = (srcbuf0, srcbuf1)
    dstbuf = (dstbuf0, dstbuf1)
    relbuf = (relbuf0, relbuf1)
    bbuf = (bbuf0, bbuf1)
    ibuf = (ibuf0, ibuf1)
    Tbuf = (Tbuf0, Tbuf1)
    isem = (isem0, isem1)
    bsem = (bsem0, bsem1)
    risem = (risem0, risem1)
    gsem = (gsem0, gsem1)
    wsem = (wsem0, wsem1)

    pltpu.sync_copy(as_hbm, as_v)
    pltpu.sync_copy(ad_hbm, ad_v)
    pltpu.sync_copy(ar_hbm, ar_v)

    @pl.when(sid == 0)
    def _():
        @pl.loop(0, 2000 // L)
        def _z(i):
            zbuf[pl.ds(i * L, L)] = jnp.zeros((L,), _f32)

        for j in range(N // 2000):
            pltpu.sync_copy(zbuf, acc_sh.at[pl.ds(j * 2000, 2000)])

    plsc.subcore_barrier()

    def fire_sidx(p, c):
        off = base + c * CH
        pltpu.async_copy(src_hbm.at[pl.ds(off, CH)], srcbuf[p], isem[p])
        pltpu.async_copy(dst_hbm.at[pl.ds(off, CH)], dstbuf[p], isem[p])
        pltpu.async_copy(rel_hbm.at[pl.ds(off, CH)], relbuf[p], isem[p])

    def wait_sidx(p, c):
        off = base + c * CH
        pltpu.make_async_copy(src_hbm.at[pl.ds(off, CH)], srcbuf[p],
                              isem[p]).wait()
        pltpu.make_async_copy(dst_hbm.at[pl.ds(off, CH)], dstbuf[p],
                              isem[p]).wait()
        pltpu.make_async_copy(rel_hbm.at[pl.ds(off, CH)], relbuf[p],
                              isem[p]).wait()

    def wait_bwrite(p, c):
        off = base + c * CH
        pltpu.make_async_copy(bbuf[p], b_hbm.at[pl.ds(off, CH)],
                              bsem[p]).wait()

    def fire_ridx(p, c):
        pltpu.async_copy(rel_hbm.at[pl.ds(base + c * CH, CH)],
                         ibuf[p], risem[p])

    def wait_ridx(p, c):
        pltpu.make_async_copy(rel_hbm.at[pl.ds(base + c * CH, CH)],
                              ibuf[p], risem[p]).wait()

    def wait_rwrite(p, c):
        pltpu.make_async_copy(Tbuf[p],
                              relout_hbm.at[pl.ds(base + c * CH, CH)],
                              wsem[p]).wait()

    def handle(p, c, last):
        np_ = 1 - p
        if not last:
            wait_ridx(np_, c + 1)

            @pl.when(c >= 1)
            def _():
                wait_rwrite(np_, c - 1)

            pltpu.async_copy(t2_hbm.at[ibuf[np_]], Tbuf[np_], gsem[np_])
            wait_sidx(np_, c + 1)

        @pl.when(c >= 2)
        def _():
            wait_bwrite(p, c - 2)

        @pl.loop(0, CH // L)
        def _vec(i):
            sl = pl.ds(i * L, L)
            sc = (plsc.load_gather(as_v, [srcbuf[p][sl]])
                  + plsc.load_gather(ad_v, [dstbuf[p][sl]])
                  + plsc.load_gather(ar_v, [relbuf[p][sl]]))
            bbuf[p][sl] = jnp.exp(jnp.where(sc >= 0.0, sc, sc * 0.01))

        pltpu.async_copy(bbuf[p], b_hbm.at[pl.ds(base + c * CH, CH)], bsem[p])
        pltpu.sync_copy(bbuf[p], acc_sh.at[srcbuf[p]], add=True)

        @pl.when(c + 2 < _NCH1)
        def _():
            fire_sidx(p, c + 2)

        pltpu.make_async_copy(t2_hbm.at[ibuf[p]], Tbuf[p], gsem[p]).wait()
        if last:
            pltpu.sync_copy(Tbuf[p], relout_hbm.at[pl.ds(base + c * CH, CH)])
        else:
            pltpu.async_copy(Tbuf[p],
                             relout_hbm.at[pl.ds(base + c * CH, CH)], wsem[p])

            @pl.when(c + 2 < _NCH1)
            def _():
                fire_ridx(p, c + 2)

    fire_sidx(0, 0)
    fire_ridx(0, 0)
    wait_sidx(0, 0)
    wait_ridx(0, 0)
    pltpu.async_copy(t2_hbm.at[ibuf0], Tbuf0, gsem0)
    fire_sidx(1, 1)
    fire_ridx(1, 1)

    @pl.loop(0, _NCH1 - 1, step=2)
    def _pair(ci):
        handle(0, ci, False)
        handle(1, ci + 1, False)

    c_last = _NCH1 - 1
    wait_rwrite(1, c_last - 1)
    handle(0, c_last, True)
    wait_bwrite(1, c_last - 1)
    wait_bwrite(0, c_last)

    plsc.subcore_barrier()

    @pl.when(sid == 0)
    def _():
        for j in range(N // 2000):
            pltpu.sync_copy(acc_sh.at[pl.ds(j * 2000, 2000)], zbuf)
            pltpu.sync_copy(zbuf, bnode2_hbm.at[pl.ds(cid * N + j * 2000, 2000)])


def _sc1(src, dst, rel, a_s, a_d, a_r, t2):
    mesh = plsc.VectorSubcoreMesh(core_axis_name="c", subcore_axis_name="s")
    f = pl.kernel(
        _sc1_body,
        out_type=[
            jax.ShapeDtypeStruct((E,), _f32),
            jax.ShapeDtypeStruct((NC * N,), _f32),
            jax.ShapeDtypeStruct((E, OUT), _f32),
        ],
        mesh=mesh,
        scratch_types=[
            pltpu.VMEM((N,), _f32),
            pltpu.VMEM((N,), _f32),
            pltpu.VMEM((N,), _f32),
            pltpu.VMEM((CH,), jnp.int32),
            pltpu.VMEM((CH,), jnp.int32),
            pltpu.VMEM((CH,), jnp.int32),
            pltpu.VMEM((CH,), jnp.int32),
            pltpu.VMEM((CH,), jnp.int32),
            pltpu.VMEM((CH,), jnp.int32),
            pltpu.VMEM((CH,), _f32),
            pltpu.VMEM((CH,), _f32),
            pltpu.VMEM((2000,), _f32),
            pltpu.VMEM((CH,), jnp.int32),
            pltpu.VMEM((CH,), jnp.int32),
            pltpu.VMEM((CH, OUT), _f32),
            pltpu.VMEM((CH, OUT), _f32),
            pltpu.SemaphoreType.DMA,
            pltpu.SemaphoreType.DMA,
            pltpu.SemaphoreType.DMA,
            pltpu.SemaphoreType.DMA,
            pltpu.SemaphoreType.DMA,
            pltpu.SemaphoreType.DMA,
            pltpu.SemaphoreType.DMA,
            pltpu.SemaphoreType.DMA,
            pltpu.SemaphoreType.DMA,
            pltpu.SemaphoreType.DMA,
            pltpu.VMEM_SHARED((N,), _f32),
        ],
        compiler_params=pltpu.CompilerParams(needs_layout_passes=False),
    )
    return f(src, dst, rel, a_s, a_d, a_r, t2)


_EPT2 = E // NS
_NCH2 = _EPT2 // CH
_WBT = 10
_RPT = N // _WBT
_WB = 40


def _sc2_body(src_hbm, dst_hbm, rel_hbm, b_hbm, bflat_hbm, rflat_hbm,
              s_hbm,
              srcbuf0, srcbuf1, dstbuf0, dstbuf1, relbuf0, relbuf1,
              bbuf0, bbuf1, sbuf0, sbuf1,
              Bbuf0, Bbuf1, Rbuf0, Rbuf1,
              isem0, isem1, Bsem0, Bsem1, Rsem0, Rsem1, ssem0, ssem1,
              acc_sh):
    cid = lax.axis_index("c")
    sid = lax.axis_index("s")
    srcbuf = (srcbuf0, srcbuf1)
    dstbuf = (dstbuf0, dstbuf1)
    relbuf = (relbuf0, relbuf1)
    bbuf = (bbuf0, bbuf1)
    sbuf = (sbuf0, sbuf1)
    Bbuf = (Bbuf0, Bbuf1)
    Rbuf = (Rbuf0, Rbuf1)
    isem = (isem0, isem1)
    Bsem = (Bsem0, Bsem1)
    Rsem = (Rsem0, Rsem1)
    ssem = (ssem0, ssem1)
    tab_off = cid * N
    ebase = sid * _EPT2

    @pl.when(sid < _WBT)
    def _():
        @pl.loop(0, _WB)
        def _z(r):
            for j in range(HALF // L):
                Bbuf0[r, pl.ds(j * L, L)] = jnp.zeros((L,), _f32)

        for j in range(_RPT // _WB):
            pltpu.sync_copy(Bbuf0.at[pl.ds(0, _WB)],
                            acc_sh.at[pl.ds(sid * _RPT + j * _WB, _WB)])

    plsc.subcore_barrier()

    def fire_idx(p, c):
        off = ebase + c * CH
        pltpu.async_copy(src_hbm.at[pl.ds(off, CH)], srcbuf[p], isem[p])
        pltpu.async_copy(dst_hbm.at[pl.ds(off, CH)], dstbuf[p], isem[p])
        pltpu.async_copy(rel_hbm.at[pl.ds(off, CH)], relbuf[p], isem[p])
        pltpu.async_copy(b_hbm.at[pl.ds(off, CH)], bbuf[p], isem[p])

    def wait_idx(p, c):
        off = ebase + c * CH
        pltpu.make_async_copy(src_hbm.at[pl.ds(off, CH)], srcbuf[p],
                              isem[p]).wait()
        pltpu.make_async_copy(dst_hbm.at[pl.ds(off, CH)], dstbuf[p],
                              isem[p]).wait()
        pltpu.make_async_copy(rel_hbm.at[pl.ds(off, CH)], relbuf[p],
                              isem[p]).wait()
        pltpu.make_async_copy(b_hbm.at[pl.ds(off, CH)], bbuf[p],
                              isem[p]).wait()

    def fire_gather(p):
        @pl.loop(0, CH // L)
        def _mkidx(i):
            sl = pl.ds(i * L, L)
            dstbuf[p][sl] = dstbuf[p][sl] + tab_off
            relbuf[p][sl] = relbuf[p][sl] + tab_off

        pltpu.async_copy(bflat_hbm.at[dstbuf[p]], Bbuf[p], Bsem[p])
        pltpu.async_copy(rflat_hbm.at[relbuf[p]], Rbuf[p], Rsem[p])

    def wait_gather(p):
        pltpu.make_async_copy(bflat_hbm.at[dstbuf[p]], Bbuf[p],
                              Bsem[p]).wait()
        pltpu.make_async_copy(rflat_hbm.at[relbuf[p]], Rbuf[p],
                              Rsem[p]).wait()

    def wait_scatter(p):
        pltpu.make_async_copy(Bbuf[p], acc_sh.at[sbuf[p]], ssem[p]).wait()

    def consume(p):
        @pl.loop(0, CH // L)
        def _scale(g):
            w16 = bbuf[p][pl.ds(g * L, L)]
            for k in range(L):
                r = g * L + k
                wk = w16[k]
                for j in range(HALF // L):
                    sl = pl.ds(j * L, L)
                    Bbuf[p][r, sl] = wk * (Bbuf[p][r, sl] + Rbuf[p][r, sl])

        @pl.loop(0, CH // L)
        def _cpy(i):
            sl = pl.ds(i * L, L)
            sbuf[p][sl] = srcbuf[p][sl]

        pltpu.async_copy(Bbuf[p], acc_sh.at[sbuf[p]], ssem[p], add=True)

    def handle(p, c, has_next=True, has_next2=True):
        np_ = 1 - p
        if has_next:
            wait_idx(np_, c + 1)

            @pl.when(c >= 1)
            def _():
                wait_scatter(np_)

            fire_gather(np_)
        else:
            wait_scatter(np_)
        wait_gather(p)
        consume(p)
        if has_next2:
            fire_idx(p, c + 2)

    fire_idx(0, 0)
    wait_idx(0, 0)
    fire_gather(0)
    fire_idx(1, 1)

    @pl.loop(0, _NCH2 - 2, step=2)
    def _pair(ci):
        handle(0, ci)
        handle(1, ci + 1)

    handle(0, _NCH2 - 2, has_next2=False)
    handle(1, _NCH2 - 1, has_next=False, has_next2=False)
    wait_scatter(1)

    plsc.subcore_barrier()

    @pl.when(sid < _WBT)
    def _():
        for j in range(_RPT // _WB):
            row = sid * _RPT + j * _WB
            pltpu.sync_copy(acc_sh.at[pl.ds(row, _WB)], Bbuf0.at[pl.ds(0, _WB)])
            pltpu.sync_copy(Bbuf0.at[pl.ds(0, _WB)],
                            s_hbm.at[cid, pl.ds(row, _WB)])


def _sc2(src, dst, rel, b, bflat, rflat):
    mesh = plsc.VectorSubcoreMesh(core_axis_name="c", subcore_axis_name="s")
    f = pl.kernel(
        _sc2_body,
        out_type=[
            jax.ShapeDtypeStruct((NC, N, HALF), _f32),
        ],
        mesh=mesh,
        scratch_types=(
            [pltpu.VMEM((CH,), jnp.int32)] * 6
            + [pltpu.VMEM((CH,), _f32)] * 2
            + [pltpu.VMEM((CH,), jnp.int32)] * 2
            + [pltpu.VMEM((CH, HALF), _f32)] * 4
            + [pltpu.SemaphoreType.DMA] * 8
            + [pltpu.VMEM_SHARED((N, HALF), _f32)]
        ),
        compiler_params=pltpu.CompilerParams(needs_layout_passes=False),
    )
    return f(src, dst, rel, b, bflat, rflat)[0]


def _fin_body(h0_ref, s0_ref, s1_ref, bn0_ref, bn1_ref, h_ref):
    bn = bn0_ref[...] + bn1_ref[...]
    s = jnp.concatenate([s0_ref[0], s1_ref[0]], axis=1)
    h_ref[...] = jnp.where(bn > 0.0,
                           h0_ref[...] + s / jnp.maximum(bn, 1e-30), 0.0)


def _finalize(h0, s, bnode2):
    nb = N // ROW_BLK
    bn_r = bnode2.reshape(NC * N, 1)
    return pl.pallas_call(
        _fin_body,
        grid=(nb,),
        in_specs=[
            pl.BlockSpec((ROW_BLK, OUT), lambda i: (i, 0)),
            pl.BlockSpec((1, ROW_BLK, HALF), lambda i: (0, i, 0)),
            pl.BlockSpec((1, ROW_BLK, HALF), lambda i: (1, i, 0)),
            pl.BlockSpec((ROW_BLK, 1), lambda i: (i, 0)),
            pl.BlockSpec((ROW_BLK, 1), lambda i: (nb + i, 0)),
        ],
        out_specs=pl.BlockSpec((ROW_BLK, OUT), lambda i: (i, 0)),
        out_shape=jax.ShapeDtypeStruct((N, OUT), _f32),
    )(h0, s, s, bn_r, bn_r)


def kernel(triplets, ent_embed, rel_embed, W_ent, b_ent, W_rel, b_rel,
           W_rel2, b_rel2, W_a, b_a, W_fc, b_fc):
    src = triplets[:, 0]
    dst = triplets[:, 1]
    rel = triplets[:, 2]

    h0, bh, rh, t2, a3 = _tables(ent_embed, rel_embed, W_ent, b_ent, W_rel,
                                 b_rel, W_rel2, b_rel2, W_a, b_a, W_fc, b_fc)

    b, bnode2, rel_out = _sc1(src, dst, rel, a3[:, 0], a3[:, 1], a3[:, 2], t2)

    s = _sc2(src, dst, rel, b,
             bh.reshape(NC * N, HALF), rh.reshape(NC * N, HALF))

    h = _finalize(h0, s, bnode2)
    return (h, rel_out)
